# baseline (speedup 1.0000x reference)
import functools

import jax
import jax.numpy as jnp
from jax import lax
from jax.experimental import pallas as pl
from jax.experimental.pallas import tpu as pltpu
from jax.experimental.pallas import tpu_sc as plsc

NUM_TABLES = 9
VOCAB = 100
HIDDEN = 64
LANES = 16
WPR = HIDDEN // 2


SBATCH = 40


def _sc_body(num_workers, edges_per_worker, chunk, num_chunks,
             attr_hbm, table_hbm, out_hbm, table_v, attr0, attr1, out0, out1,
             sem_a0, sem_a1, sem_o0, sem_o1):
  num_cores = num_workers // 16
  wid = lax.axis_index("s") * num_cores + lax.axis_index("c")

  pltpu.sync_copy(table_hbm, table_v)

  def attr_copy(g, buf, sem):
    base = wid * edges_per_worker + g * chunk
    return pltpu.make_async_copy(
        attr_hbm.at[pl.ds(base * NUM_TABLES, chunk * NUM_TABLES)],
        buf.at[pl.ds(0, chunk * NUM_TABLES)], sem)

  def out_copy(g, buf, sem):
    base = wid * edges_per_worker + g * chunk
    return pltpu.make_async_copy(
        buf, out_hbm.at[pl.ds(base * HIDDEN, chunk * HIDDEN)], sem)

  def compute(attr_v, out_v):
    @plsc.parallel_loop(0, chunk, 1, unroll=4)
    def edge_body(e):
      av = attr_v[pl.ds(e * NUM_TABLES, LANES)]
      accs = [None] * 4
      for i in range(NUM_TABLES):
        rowb = (av[i] + VOCAB * i) * WPR
        for h in range(2):
          iw = table_v[pl.ds(rowb + LANES * h, LANES)]
          lo = plsc.bitcast(iw << 16, jnp.float32)
          hi = plsc.bitcast(iw & jnp.int32(-65536), jnp.float32)
          if i == 0:
            accs[2 * h] = lo
            accs[2 * h + 1] = hi
          else:
            accs[2 * h] = accs[2 * h] + lo
            accs[2 * h + 1] = accs[2 * h + 1] + hi
      for cb in range(4):
        out_v[pl.ds(e * HIDDEN + 16 * cb, LANES)] = accs[cb]

  num_pairs = num_chunks // 2
  attr_copy(0, attr0, sem_a0).start()

  def pair_body(gg, carry):
    g0 = gg * 2
    g1 = g0 + 1

    attr_copy(g0, attr0, sem_a0).wait()
    attr_copy(g1, attr1, sem_a1).start()

    @pl.when(gg > 0)
    def _():
      out_copy(g0 - 2, out0, sem_o0).wait()

    compute(attr0, out0)
    out_copy(g0, out0, sem_o0).start()

    attr_copy(g1, attr1, sem_a1).wait()
    attr_copy(g1 + 1, attr0, sem_a0).start()

    @pl.when(gg > 0)
    def _():
      out_copy(g1 - 2, out1, sem_o1).wait()

    compute(attr1, out1)
    out_copy(g1, out1, sem_o1).start()
    return carry

  lax.fori_loop(0, num_pairs, pair_body, 0)

  gt = num_chunks - 1
  attr_copy(gt, attr0, sem_a0).wait()
  out_copy(gt - 2, out0, sem_o0).wait()
  compute(attr0, out0)
  out_copy(gt, out0, sem_o0).start()
  out_copy(gt, out0, sem_o0).wait()
  out_copy(gt - 1, out1, sem_o1).wait()


@jax.jit
def kernel(edge_attr, tables):
  e_total = edge_attr.shape[0]
  info = plsc.get_sparse_core_info()
  num_workers = info.num_cores * info.num_subcores
  assert e_total % num_workers == 0
  edges_per_worker = e_total // num_workers
  chunk = 200
  assert edges_per_worker % chunk == 0
  num_chunks = edges_per_worker // chunk
  assert num_chunks % 2 == 1 and num_chunks > 2

  attr = edge_attr.astype(jnp.int32).reshape(-1)
  tab = tables.astype(jnp.float32).reshape(NUM_TABLES * VOCAB, 2, 2, LANES)
  tab = tab.transpose(0, 1, 3, 2).reshape(-1).astype(jnp.bfloat16)
  tab = jax.lax.bitcast_convert_type(tab.reshape(-1, 2), jnp.int32)

  mesh = plsc.VectorSubcoreMesh(core_axis_name="c", subcore_axis_name="s")
  call = pl.kernel(
      functools.partial(_sc_body, num_workers, edges_per_worker, chunk,
                        num_chunks),
      out_type=jax.ShapeDtypeStruct((e_total * HIDDEN,), jnp.float32),
      mesh=mesh,
      compiler_params=pltpu.CompilerParams(needs_layout_passes=False),
      scratch_types=[
          pltpu.VMEM((NUM_TABLES * VOCAB * WPR,), jnp.int32),
          pltpu.VMEM((chunk * NUM_TABLES + LANES,), jnp.int32),
          pltpu.VMEM((chunk * NUM_TABLES + LANES,), jnp.int32),
          pltpu.VMEM((chunk * HIDDEN,), jnp.float32),
          pltpu.VMEM((chunk * HIDDEN,), jnp.float32),
          pltpu.SemaphoreType.DMA,
          pltpu.SemaphoreType.DMA,
          pltpu.SemaphoreType.DMA,
          pltpu.SemaphoreType.DMA,
      ],
  )
  out = call(attr, tab)
  return out.reshape(e_total, HIDDEN)


# speedup vs baseline: 1.0959x; 1.0959x over previous
import functools

import jax
import jax.numpy as jnp
from jax import lax
from jax.experimental import pallas as pl
from jax.experimental.pallas import tpu as pltpu
from jax.experimental.pallas import tpu_sc as plsc

NUM_TABLES = 9
VOCAB = 100
HIDDEN = 64
LANES = 16
WPR = HIDDEN // 2


SBATCH = 40


def _sc_body(num_workers, edges_per_worker, chunk, num_chunks,
             attr_hbm, table_hbm, out_hbm, table_v, attr0, attr1, out0, out1,
             sem_a0, sem_a1, sem_o0, sem_o1):
  num_cores = num_workers // 16
  wid = lax.axis_index("s") * num_cores + lax.axis_index("c")

  pltpu.sync_copy(table_hbm, table_v)

  def attr_copy(g, buf, sem):
    base = wid * edges_per_worker + g * chunk
    return pltpu.make_async_copy(
        attr_hbm.at[pl.ds(base * NUM_TABLES, chunk * NUM_TABLES)],
        buf.at[pl.ds(0, chunk * NUM_TABLES)], sem)

  def out_copy(g, buf, sem):
    base = wid * edges_per_worker + g * chunk
    return pltpu.make_async_copy(
        buf, out_hbm.at[pl.ds(base * HIDDEN, chunk * HIDDEN)], sem)

  def compute(attr_v, out_v):
    @plsc.parallel_loop(0, chunk, 1, unroll=2)
    def edge_body(e):
      av = attr_v[pl.ds(e * NUM_TABLES, LANES)]
      accs = [None, None]
      for i in range(NUM_TABLES):
        rowb = (av[i] + VOCAB * i) * WPR
        for h in range(2):
          iw = table_v[pl.ds(rowb + LANES * h, LANES)]
          bw = plsc.bitcast(iw, jnp.bfloat16)
          accs[h] = bw if i == 0 else accs[h] + bw
      for h in range(2):
        iw = plsc.bitcast(accs[h], jnp.int32)
        lo = plsc.bitcast(iw << 16, jnp.float32)
        hi = plsc.bitcast(iw & jnp.int32(-65536), jnp.float32)
        out_v[pl.ds(e * HIDDEN + 32 * h, LANES)] = lo
        out_v[pl.ds(e * HIDDEN + 32 * h + LANES, LANES)] = hi

  num_pairs = num_chunks // 2
  attr_copy(0, attr0, sem_a0).start()

  def pair_body(gg, carry):
    g0 = gg * 2
    g1 = g0 + 1

    attr_copy(g0, attr0, sem_a0).wait()
    attr_copy(g1, attr1, sem_a1).start()

    @pl.when(gg > 0)
    def _():
      out_copy(g0 - 2, out0, sem_o0).wait()

    compute(attr0, out0)
    out_copy(g0, out0, sem_o0).start()

    attr_copy(g1, attr1, sem_a1).wait()
    attr_copy(g1 + 1, attr0, sem_a0).start()

    @pl.when(gg > 0)
    def _():
      out_copy(g1 - 2, out1, sem_o1).wait()

    compute(attr1, out1)
    out_copy(g1, out1, sem_o1).start()
    return carry

  lax.fori_loop(0, num_pairs, pair_body, 0)

  gt = num_chunks - 1
  attr_copy(gt, attr0, sem_a0).wait()
  out_copy(gt - 2, out0, sem_o0).wait()
  compute(attr0, out0)
  out_copy(gt, out0, sem_o0).start()
  out_copy(gt, out0, sem_o0).wait()
  out_copy(gt - 1, out1, sem_o1).wait()


@jax.jit
def kernel(edge_attr, tables):
  e_total = edge_attr.shape[0]
  info = plsc.get_sparse_core_info()
  num_workers = info.num_cores * info.num_subcores
  assert e_total % num_workers == 0
  edges_per_worker = e_total // num_workers
  chunk = 200
  assert edges_per_worker % chunk == 0
  num_chunks = edges_per_worker // chunk
  assert num_chunks % 2 == 1 and num_chunks > 2

  attr = edge_attr.astype(jnp.int32).reshape(-1)
  tab = tables.astype(jnp.float32).reshape(NUM_TABLES * VOCAB, 2, 2, LANES)
  tab = tab.transpose(0, 1, 3, 2).reshape(-1).astype(jnp.bfloat16)
  tab = jax.lax.bitcast_convert_type(tab.reshape(-1, 2), jnp.int32)

  mesh = plsc.VectorSubcoreMesh(core_axis_name="c", subcore_axis_name="s")
  call = pl.kernel(
      functools.partial(_sc_body, num_workers, edges_per_worker, chunk,
                        num_chunks),
      out_type=jax.ShapeDtypeStruct((e_total * HIDDEN,), jnp.float32),
      mesh=mesh,
      compiler_params=pltpu.CompilerParams(needs_layout_passes=False),
      scratch_types=[
          pltpu.VMEM((NUM_TABLES * VOCAB * WPR,), jnp.int32),
          pltpu.VMEM((chunk * NUM_TABLES + LANES,), jnp.int32),
          pltpu.VMEM((chunk * NUM_TABLES + LANES,), jnp.int32),
          pltpu.VMEM((chunk * HIDDEN,), jnp.float32),
          pltpu.VMEM((chunk * HIDDEN,), jnp.float32),
          pltpu.SemaphoreType.DMA,
          pltpu.SemaphoreType.DMA,
          pltpu.SemaphoreType.DMA,
          pltpu.SemaphoreType.DMA,
      ],
  )
  return call(attr, tab).reshape(e_total, HIDDEN)


# speedup vs baseline: 1.0965x; 1.0005x over previous
import functools

import jax
import jax.numpy as jnp
from jax import lax
from jax.experimental import pallas as pl
from jax.experimental.pallas import tpu as pltpu
from jax.experimental.pallas import tpu_sc as plsc

NUM_TABLES = 9
VOCAB = 100
HIDDEN = 64
LANES = 16
WPR = HIDDEN // 2


def _sc_body(num_workers, edges_per_worker, chunk, num_chunks,
             attr_hbm, table_hbm, out_hbm, table_v, attr0, attr1, out0, out1,
             sem_a0, sem_a1, sem_o0, sem_o1):
  num_cores = num_workers // 16
  wid = lax.axis_index("s") * num_cores + lax.axis_index("c")

  pltpu.sync_copy(table_hbm, table_v)

  def attr_copy(g, buf, sem):
    base = wid * edges_per_worker + g * chunk
    return pltpu.make_async_copy(
        attr_hbm.at[pl.ds(base * NUM_TABLES, chunk * NUM_TABLES)],
        buf.at[pl.ds(0, chunk * NUM_TABLES)], sem)

  def out_copy(g, buf, sem):
    base = wid * edges_per_worker + g * chunk
    return pltpu.make_async_copy(
        buf, out_hbm.at[pl.ds(base * HIDDEN, chunk * HIDDEN)], sem)

  def compute(attr_v, out_v):
    @plsc.parallel_loop(0, chunk, 1, unroll=2)
    def edge_body(e):
      av = attr_v[pl.ds(e * NUM_TABLES, LANES)]
      accs = [None, None]
      for i in range(NUM_TABLES):
        rowb = (av[i] + VOCAB * i) * WPR
        for h in range(2):
          iw = table_v[pl.ds(rowb + LANES * h, LANES)]
          bw = plsc.bitcast(iw, jnp.bfloat16)
          accs[h] = bw if i == 0 else accs[h] + bw
      for h in range(2):
        iw = plsc.bitcast(accs[h], jnp.int32)
        lo = plsc.bitcast(iw << 16, jnp.float32)
        hi = plsc.bitcast(iw & jnp.int32(-65536), jnp.float32)
        out_v[pl.ds(e * HIDDEN + 32 * h, LANES)] = lo
        out_v[pl.ds(e * HIDDEN + 32 * h + LANES, LANES)] = hi

  num_pairs = num_chunks // 2
  attr_copy(0, attr0, sem_a0).start()

  def pair_body(gg, carry):
    g0 = gg * 2
    g1 = g0 + 1

    attr_copy(g0, attr0, sem_a0).wait()
    attr_copy(g1, attr1, sem_a1).start()

    @pl.when(gg > 0)
    def _():
      out_copy(g0 - 2, out0, sem_o0).wait()

    compute(attr0, out0)
    out_copy(g0, out0, sem_o0).start()

    attr_copy(g1, attr1, sem_a1).wait()
    attr_copy(g1 + 1, attr0, sem_a0).start()

    @pl.when(gg > 0)
    def _():
      out_copy(g1 - 2, out1, sem_o1).wait()

    compute(attr1, out1)
    out_copy(g1, out1, sem_o1).start()
    return carry

  lax.fori_loop(0, num_pairs, pair_body, 0)

  gt = num_chunks - 1
  attr_copy(gt, attr0, sem_a0).wait()
  out_copy(gt - 2, out0, sem_o0).wait()
  compute(attr0, out0)
  out_copy(gt, out0, sem_o0).start()
  out_copy(gt, out0, sem_o0).wait()
  out_copy(gt - 1, out1, sem_o1).wait()


@jax.jit
def kernel(edge_attr, tables):
  e_total = edge_attr.shape[0]
  info = plsc.get_sparse_core_info()
  num_workers = info.num_cores * info.num_subcores
  assert e_total % num_workers == 0
  edges_per_worker = e_total // num_workers
  chunk = 200
  assert edges_per_worker % chunk == 0
  num_chunks = edges_per_worker // chunk
  assert num_chunks % 2 == 1 and num_chunks > 2

  attr = edge_attr.astype(jnp.int32).reshape(-1)
  tab = tables.astype(jnp.float32).reshape(NUM_TABLES * VOCAB, 2, 2, LANES)
  tab = tab.transpose(0, 1, 3, 2).reshape(-1).astype(jnp.bfloat16)
  tab = jax.lax.bitcast_convert_type(tab.reshape(-1, 2), jnp.int32)

  mesh = plsc.VectorSubcoreMesh(core_axis_name="c", subcore_axis_name="s")
  call = pl.kernel(
      functools.partial(_sc_body, num_workers, edges_per_worker, chunk,
                        num_chunks),
      out_type=jax.ShapeDtypeStruct((e_total * HIDDEN,), jnp.float32),
      mesh=mesh,
      compiler_params=pltpu.CompilerParams(needs_layout_passes=False),
      scratch_types=[
          pltpu.VMEM((NUM_TABLES * VOCAB * WPR,), jnp.int32),
          pltpu.VMEM((chunk * NUM_TABLES + LANES,), jnp.int32),
          pltpu.VMEM((chunk * NUM_TABLES + LANES,), jnp.int32),
          pltpu.VMEM((chunk * HIDDEN,), jnp.float32),
          pltpu.VMEM((chunk * HIDDEN,), jnp.float32),
          pltpu.SemaphoreType.DMA,
          pltpu.SemaphoreType.DMA,
          pltpu.SemaphoreType.DMA,
          pltpu.SemaphoreType.DMA,
      ],
  )
  return call(attr, tab).reshape(e_total, HIDDEN)
